# baseline (speedup 1.0000x reference)
import functools

import jax
import jax.numpy as jnp
from jax import lax
from jax.experimental import pallas as pl
from jax.experimental.pallas import tpu as pltpu
from jax.experimental.pallas import tpu_sc as plsc

D = 64
DP = 128
NEG = 10
NC = 2
NS = 16
NW = NC * NS
CHA = 32
SUB = 32


def _plain_gathers(wid, base, steps, gbuf, sems):
    def fire(i):
        table, idx, _, c = steps[i]
        return pltpu.async_copy(table.at[idx.at[c]], gbuf.at[i % 2],
                                sems[i % 2])
    cp = fire(0)
    for i in range(len(steps)):
        nxt = fire(i + 1) if i + 1 < len(steps) else None
        cp.wait()
        _, _, out, c = steps[i]
        pltpu.sync_copy(gbuf.at[i % 2], out.at[pl.ds(base + c * CHA, CHA)])
        cp = nxt


def _sc_k1_body(nodes_u, nodes_v, pot_idx, delta, u_emb, v_emb,
                out_uid, out_vmix, out_pmean,
                idx_u, idx_v, pidx, dbuf, gbuf, vbuf, pbuf, psub, vmsub,
                sem0, sem1):
    wid = lax.axis_index("s") * NC + lax.axis_index("c")
    n_ch = idx_u.shape[0]
    bpw = n_ch * CHA
    base = wid * bpw
    sems = (sem0, sem1)

    pltpu.sync_copy(nodes_u.at[pl.ds(wid * n_ch, n_ch)], idx_u)
    pltpu.sync_copy(nodes_v.at[pl.ds(wid * n_ch, n_ch)], idx_v)
    pltpu.sync_copy(pot_idx.at[pl.ds(wid * n_ch * NEG, n_ch * NEG)], pidx)
    pltpu.sync_copy(delta.at[pl.ds(base, bpw)], dbuf)

    _plain_gathers(wid, base,
                   [(u_emb, idx_u, out_uid, c) for c in range(n_ch)],
                   gbuf, sems)

    n_sub = bpw // SUB

    def fire_b(s):
        k = s % 2
        cps = [pltpu.async_copy(v_emb.at[pidx.at[s * NEG + g]],
                                pbuf.at[k, pl.ds(g * SUB, SUB)], sems[k])
               for g in range(NEG)]
        cps.append(pltpu.async_copy(v_emb.at[idx_v.at[s]], vbuf.at[k],
                                    sems[k]))
        return cps

    cps = fire_b(0)
    for s in range(n_sub):
        nxt = fire_b(s + 1) if s + 1 < n_sub else None
        for cp in cps:
            cp.wait()
        k = s % 2

        def body(r, carry):
            dl = dbuf[s * SUB + r, :]
            one_m = jnp.float32(1.0) - dl
            for grp in range(D // 16):
                sl = pl.ds(grp * 16, 16)
                acc = pbuf[k, r * NEG, sl]
                for j in range(1, NEG):
                    acc = acc + pbuf[k, r * NEG + j, sl]
                m = acc / jnp.float32(NEG)
                psub[r, sl] = m
                vmsub[r, sl] = dl * vbuf[k, r, sl] + one_m * m
            return carry

        lax.fori_loop(0, SUB, body, jnp.int32(0))

        pltpu.sync_copy(psub, out_pmean.at[pl.ds(base + s * SUB, SUB)])
        pltpu.sync_copy(vmsub, out_vmix.at[pl.ds(base + s * SUB, SUB)])
        cps = nxt


def _sc_k2_body(nodes_u, nodes_v, protos, u_review, v_review,
                out_protos, out_urev, out_vrev,
                idx_u, idx_v, gbuf, sem0, sem1):
    wid = lax.axis_index("s") * NC + lax.axis_index("c")
    n_ch = idx_u.shape[0]
    bpw = n_ch * CHA
    base = wid * bpw

    pltpu.sync_copy(nodes_u.at[pl.ds(wid * n_ch, n_ch)], idx_u)
    pltpu.sync_copy(nodes_v.at[pl.ds(wid * n_ch, n_ch)], idx_v)

    steps = [(table, idx, out, c)
             for table, idx, out in ((u_review, idx_u, out_urev),
                                     (protos, idx_u, out_protos),
                                     (v_review, idx_v, out_vrev))
             for c in range(n_ch)]
    _plain_gathers(wid, base, steps, gbuf, (sem0, sem1))


def _make_sc_k1(B):
    n_ch = B // (NW * CHA)
    out = jax.ShapeDtypeStruct((B, DP), jnp.float32)
    return pl.kernel(
        _sc_k1_body,
        out_type=(out,) * 3,
        mesh=plsc.VectorSubcoreMesh(core_axis_name="c", subcore_axis_name="s"),
        scratch_types=[
            pltpu.VMEM((n_ch, CHA), jnp.int32),
            pltpu.VMEM((n_ch, CHA), jnp.int32),
            pltpu.VMEM((n_ch * NEG, CHA), jnp.int32),
            pltpu.VMEM((n_ch * CHA, 16), jnp.float32),
            pltpu.VMEM((2, CHA, DP), jnp.float32),
            pltpu.VMEM((2, SUB, DP), jnp.float32),
            pltpu.VMEM((2, SUB * NEG, DP), jnp.float32),
            pltpu.VMEM((SUB, DP), jnp.float32),
            pltpu.VMEM((SUB, DP), jnp.float32),
            pltpu.SemaphoreType.DMA,
            pltpu.SemaphoreType.DMA,
        ],
        compiler_params=pltpu.CompilerParams(use_tc_tiling_on_sc=False),
    )


def _make_sc_k2(B):
    n_ch = B // (NW * CHA)
    out = jax.ShapeDtypeStruct((B, DP), jnp.float32)
    return pl.kernel(
        _sc_k2_body,
        out_type=(out,) * 3,
        mesh=plsc.VectorSubcoreMesh(core_axis_name="c", subcore_axis_name="s"),
        scratch_types=[
            pltpu.VMEM((n_ch, CHA), jnp.int32),
            pltpu.VMEM((n_ch, CHA), jnp.int32),
            pltpu.VMEM((2, CHA, DP), jnp.float32),
            pltpu.SemaphoreType.DMA,
            pltpu.SemaphoreType.DMA,
        ],
        compiler_params=pltpu.CompilerParams(use_tc_tiling_on_sc=False),
    )


def _mlp_body(uid, vmix, w1u, w1v, b1, g1, be1, w2, b2, g2, be2,
              w3, b3, g3, be3, wc, bc, out):
    def bn_relu(x, gamma, beta):
        x = jnp.maximum(x, 0.0)
        m = jnp.mean(x, axis=0, keepdims=True)
        v = jnp.mean(x * x, axis=0, keepdims=True) - m * m
        s = gamma * lax.rsqrt(v + jnp.float32(1e-5))
        return x * s + (beta - m * s)

    x1 = jnp.dot(uid[:, :D], w1u[...]) + jnp.dot(vmix[:, :D], w1v[...])
    h1 = bn_relu(x1 + b1[...], g1[...], be1[...])
    h2 = bn_relu(jnp.dot(h1, w2[...]) + b2[...], g2[...], be2[...])
    h3 = bn_relu(jnp.dot(h2, w3[...]) + b3[...], g3[...], be3[...])
    logit = jnp.sum(h3 * wc[...], axis=1, keepdims=True) + bc[...]
    out[...] = jax.nn.sigmoid(logit)


def kernel(nodes_u, nodes_v, potential_items, inter_nums, delta, global_protos,
           u_emb, v_emb, u_review, v_review,
           W1, b1, g1, be1, W2, b2, g2, be2, W3, b3, g3, be3, Wc, bc):
    B = nodes_u.shape[0]
    nodes_u2 = nodes_u.reshape(B // CHA, CHA)
    nodes_v2 = nodes_v.reshape(B // CHA, CHA)
    pot2 = potential_items.reshape(B * NEG // CHA, CHA)
    delta16 = jnp.broadcast_to(delta.reshape(B, 1), (B, 16))
    zpad = jnp.broadcast_to(delta.reshape(-1)[:1] * 0.0,
                            (u_emb.shape[0], DP - D))
    pad = lambda t: jnp.concatenate([t, zpad], axis=1)

    gp_p, ur_p, vr_p = pad(global_protos), pad(u_review), pad(v_review)
    ue_p, ve_p = pad(u_emb), pad(v_emb)

    u_id_p, v_mixed_p, pmean_p = _make_sc_k1(B)(
        nodes_u2, nodes_v2, pot2, delta16, ue_p, ve_p)

    u_feats_p, u_rev_p, v_rev_p = _make_sc_k2(B)(
        nodes_u2, nodes_v2, gp_p, ur_p, vr_p)

    pred = pl.pallas_call(
        _mlp_body,
        out_shape=jax.ShapeDtypeStruct((B, 1), jnp.float32),
        compiler_params=pltpu.CompilerParams(
            vmem_limit_bytes=100 * 1024 * 1024),
    )(u_id_p, v_mixed_p, W1[:D], W1[D:], b1.reshape(1, D), g1.reshape(1, D),
      be1.reshape(1, D), W2, b2.reshape(1, D // 2), g2.reshape(1, D // 2),
      be2.reshape(1, D // 2), W3, b3.reshape(1, D // 4), g3.reshape(1, D // 4),
      be3.reshape(1, D // 4), Wc.reshape(1, D // 4), bc.reshape(1, 1))

    return (u_feats_p[:, :D], pred.reshape(B), u_id_p[:, :D],
            v_mixed_p[:, :D], u_rev_p[:, :D], v_rev_p[:, :D], pmean_p[:, :D])


# speedup vs baseline: 1.0424x; 1.0424x over previous
import functools

import jax
import jax.numpy as jnp
from jax import lax
from jax.experimental import pallas as pl
from jax.experimental.pallas import tpu as pltpu
from jax.experimental.pallas import tpu_sc as plsc

D = 64
DP = 128
NEG = 10
NC = 2
NS = 16
NW = NC * NS
CHA = 32
SUB = 32


def _plain_gathers(wid, base, steps, gbuf, sems):
    def fire(i):
        table, idx, _, c = steps[i]
        return pltpu.async_copy(table.at[idx.at[c]], gbuf.at[i % 2],
                                sems[i % 2])
    cp = fire(0)
    for i in range(len(steps)):
        nxt = fire(i + 1) if i + 1 < len(steps) else None
        cp.wait()
        _, _, out, c = steps[i]
        pltpu.sync_copy(gbuf.at[i % 2], out.at[pl.ds(base + c * CHA, CHA)])
        cp = nxt


def _sc_k1_body(nodes_u, nodes_v, pot_idx, delta, u_emb, v_emb,
                out_uid, out_vmix, out_pmean,
                idx_u, idx_v, pidx, dbuf, gbuf, vbuf, pbuf, psub, vmsub,
                sem0, sem1, sem2, sem3):
    wid = lax.axis_index("s") * NC + lax.axis_index("c")
    n_ch = idx_u.shape[0]
    bpw = n_ch * CHA
    base = wid * bpw
    sems = (sem0, sem1)
    osems = (sem2, sem3)

    pltpu.sync_copy(nodes_u.at[pl.ds(wid * n_ch, n_ch)], idx_u)
    pltpu.sync_copy(nodes_v.at[pl.ds(wid * n_ch, n_ch)], idx_v)
    pltpu.sync_copy(pot_idx.at[:, pl.ds(base, bpw)], pidx)
    pltpu.sync_copy(delta.at[pl.ds(base, bpw)], dbuf)

    _plain_gathers(wid, base,
                   [(u_emb, idx_u, out_uid, c) for c in range(n_ch)],
                   gbuf, sems)

    n_sub = bpw // SUB

    def fire_b(s):
        k = s % 2
        cps = [pltpu.async_copy(v_emb.at[pidx.at[g, pl.ds(s * SUB, SUB)]],
                                pbuf.at[k, pl.ds(g * SUB, SUB)], sems[k])
               for g in range(NEG)]
        cps.append(pltpu.async_copy(v_emb.at[idx_v.at[s]], vbuf.at[k],
                                    sems[k]))
        return cps

    cps = fire_b(0)
    ocps = [None, None]
    for s in range(n_sub):
        nxt = fire_b(s + 1) if s + 1 < n_sub else None
        for cp in cps:
            cp.wait()
        k = s % 2
        if ocps[k] is not None:
            for cp in ocps[k]:
                cp.wait()

        def body(r, carry):
            dl = dbuf[s * SUB + r, :]
            one_m = jnp.float32(1.0) - dl
            for grp in range(D // 16):
                sl = pl.ds(grp * 16, 16)
                acc = pbuf[k, r, sl]
                for j in range(1, NEG):
                    acc = acc + pbuf[k, j * SUB + r, sl]
                m = acc / jnp.float32(NEG)
                psub[k, r, sl] = m
                vmsub[k, r, sl] = dl * vbuf[k, r, sl] + one_m * m
            return carry

        lax.fori_loop(0, SUB, body, jnp.int32(0))

        ocps[k] = [
            pltpu.async_copy(psub.at[k],
                             out_pmean.at[pl.ds(base + s * SUB, SUB)],
                             osems[k]),
            pltpu.async_copy(vmsub.at[k],
                             out_vmix.at[pl.ds(base + s * SUB, SUB)],
                             osems[k]),
        ]
        cps = nxt
    for cp in ocps[0] + ocps[1]:
        cp.wait()


def _sc_k2_body(nodes_u, nodes_v, protos, u_review, v_review,
                out_protos, out_urev, out_vrev,
                idx_u, idx_v, gbuf, sem0, sem1):
    wid = lax.axis_index("s") * NC + lax.axis_index("c")
    n_ch = idx_u.shape[0]
    bpw = n_ch * CHA
    base = wid * bpw

    pltpu.sync_copy(nodes_u.at[pl.ds(wid * n_ch, n_ch)], idx_u)
    pltpu.sync_copy(nodes_v.at[pl.ds(wid * n_ch, n_ch)], idx_v)

    steps = [(table, idx, out, c)
             for table, idx, out in ((u_review, idx_u, out_urev),
                                     (protos, idx_u, out_protos),
                                     (v_review, idx_v, out_vrev))
             for c in range(n_ch)]
    _plain_gathers(wid, base, steps, gbuf, (sem0, sem1))


def _make_sc_k1(B):
    n_ch = B // (NW * CHA)
    out = jax.ShapeDtypeStruct((B, DP), jnp.float32)
    return pl.kernel(
        _sc_k1_body,
        out_type=(out,) * 3,
        mesh=plsc.VectorSubcoreMesh(core_axis_name="c", subcore_axis_name="s"),
        scratch_types=[
            pltpu.VMEM((n_ch, CHA), jnp.int32),
            pltpu.VMEM((n_ch, CHA), jnp.int32),
            pltpu.VMEM((NEG, n_ch * CHA), jnp.int32),
            pltpu.VMEM((n_ch * CHA, 16), jnp.float32),
            pltpu.VMEM((2, CHA, DP), jnp.float32),
            pltpu.VMEM((2, SUB, DP), jnp.float32),
            pltpu.VMEM((2, SUB * NEG, DP), jnp.float32),
            pltpu.VMEM((2, SUB, DP), jnp.float32),
            pltpu.VMEM((2, SUB, DP), jnp.float32),
            pltpu.SemaphoreType.DMA,
            pltpu.SemaphoreType.DMA,
            pltpu.SemaphoreType.DMA,
            pltpu.SemaphoreType.DMA,
        ],
        compiler_params=pltpu.CompilerParams(use_tc_tiling_on_sc=False),
    )


def _make_sc_k2(B):
    n_ch = B // (NW * CHA)
    out = jax.ShapeDtypeStruct((B, DP), jnp.float32)
    return pl.kernel(
        _sc_k2_body,
        out_type=(out,) * 3,
        mesh=plsc.VectorSubcoreMesh(core_axis_name="c", subcore_axis_name="s"),
        scratch_types=[
            pltpu.VMEM((n_ch, CHA), jnp.int32),
            pltpu.VMEM((n_ch, CHA), jnp.int32),
            pltpu.VMEM((2, CHA, DP), jnp.float32),
            pltpu.SemaphoreType.DMA,
            pltpu.SemaphoreType.DMA,
        ],
        compiler_params=pltpu.CompilerParams(use_tc_tiling_on_sc=False),
    )


def _mlp_body(uid, vmix, w1u, w1v, b1, g1, be1, w2, b2, g2, be2,
              w3, b3, g3, be3, wc, bc, out):
    def bn_relu(x, gamma, beta):
        x = jnp.maximum(x, 0.0)
        m = jnp.mean(x, axis=0, keepdims=True)
        v = jnp.mean(x * x, axis=0, keepdims=True) - m * m
        s = gamma * lax.rsqrt(v + jnp.float32(1e-5))
        return x * s + (beta - m * s)

    x1 = jnp.dot(uid[:, :D], w1u[...]) + jnp.dot(vmix[:, :D], w1v[...])
    h1 = bn_relu(x1 + b1[...], g1[...], be1[...])
    h2 = bn_relu(jnp.dot(h1, w2[...]) + b2[...], g2[...], be2[...])
    h3 = bn_relu(jnp.dot(h2, w3[...]) + b3[...], g3[...], be3[...])
    logit = jnp.sum(h3 * wc[...], axis=1, keepdims=True) + bc[...]
    out[...] = jax.nn.sigmoid(logit)


def kernel(nodes_u, nodes_v, potential_items, inter_nums, delta, global_protos,
           u_emb, v_emb, u_review, v_review,
           W1, b1, g1, be1, W2, b2, g2, be2, W3, b3, g3, be3, Wc, bc):
    B = nodes_u.shape[0]
    nodes_u2 = nodes_u.reshape(B // CHA, CHA)
    nodes_v2 = nodes_v.reshape(B // CHA, CHA)
    pot2 = potential_items.T
    delta16 = jnp.broadcast_to(delta.reshape(B, 1), (B, 16))
    zpad = jnp.broadcast_to(delta.reshape(-1)[:1] * 0.0,
                            (u_emb.shape[0], DP - D))
    pad = lambda t: jnp.concatenate([t, zpad], axis=1)

    gp_p, ur_p, vr_p = pad(global_protos), pad(u_review), pad(v_review)
    ue_p, ve_p = pad(u_emb), pad(v_emb)

    u_id_p, v_mixed_p, pmean_p = _make_sc_k1(B)(
        nodes_u2, nodes_v2, pot2, delta16, ue_p, ve_p)

    u_feats_p, u_rev_p, v_rev_p = _make_sc_k2(B)(
        nodes_u2, nodes_v2, gp_p, ur_p, vr_p)

    pred = pl.pallas_call(
        _mlp_body,
        out_shape=jax.ShapeDtypeStruct((B, 1), jnp.float32),
        compiler_params=pltpu.CompilerParams(
            vmem_limit_bytes=100 * 1024 * 1024),
    )(u_id_p, v_mixed_p, W1[:D], W1[D:], b1.reshape(1, D), g1.reshape(1, D),
      be1.reshape(1, D), W2, b2.reshape(1, D // 2), g2.reshape(1, D // 2),
      be2.reshape(1, D // 2), W3, b3.reshape(1, D // 4), g3.reshape(1, D // 4),
      be3.reshape(1, D // 4), Wc.reshape(1, D // 4), bc.reshape(1, 1))

    return (u_feats_p[:, :D], pred.reshape(B), u_id_p[:, :D],
            v_mixed_p[:, :D], u_rev_p[:, :D], v_rev_p[:, :D], pmean_p[:, :D])


# speedup vs baseline: 1.0448x; 1.0023x over previous
import functools

import jax
import jax.numpy as jnp
from jax import lax
from jax.experimental import pallas as pl
from jax.experimental.pallas import tpu as pltpu
from jax.experimental.pallas import tpu_sc as plsc

D = 64
DP = 128
NEG = 10
NC = 2
NS = 16
NW = NC * NS
CHA = 32
SUB = 32


def _plain_gathers(wid, base, steps, gbuf, sems, osems):
    def fire(i):
        table, idx, _, c = steps[i]
        return pltpu.async_copy(table.at[idx.at[c]], gbuf.at[i % 2],
                                sems[i % 2])
    cp = fire(0)
    ocp = [None, None]
    for i in range(len(steps)):
        nxt = None
        if i + 1 < len(steps):
            if ocp[(i + 1) % 2] is not None:
                ocp[(i + 1) % 2].wait()
                ocp[(i + 1) % 2] = None
            nxt = fire(i + 1)
        cp.wait()
        _, _, out, c = steps[i]
        ocp[i % 2] = pltpu.async_copy(
            gbuf.at[i % 2], out.at[pl.ds(base + c * CHA, CHA)], osems[i % 2])
        cp = nxt
    for o in ocp:
        if o is not None:
            o.wait()


def _sc_k1_body(nodes_u, nodes_v, pot_idx, delta, u_emb, v_emb,
                out_uid, out_vmix, out_pmean,
                idx_u, idx_v, pidx, dbuf, gbuf, vbuf, pbuf, psub, vmsub,
                sem0, sem1, sem2, sem3):
    wid = lax.axis_index("s") * NC + lax.axis_index("c")
    n_ch = idx_u.shape[0]
    bpw = n_ch * CHA
    base = wid * bpw
    sems = (sem0, sem1)
    osems = (sem2, sem3)

    pltpu.sync_copy(nodes_u.at[pl.ds(wid * n_ch, n_ch)], idx_u)
    pltpu.sync_copy(nodes_v.at[pl.ds(wid * n_ch, n_ch)], idx_v)
    pltpu.sync_copy(pot_idx.at[:, pl.ds(base, bpw)], pidx)
    pltpu.sync_copy(delta.at[pl.ds(base, bpw)], dbuf)

    _plain_gathers(wid, base,
                   [(u_emb, idx_u, out_uid, c) for c in range(n_ch)],
                   gbuf, sems, osems)

    n_sub = bpw // SUB

    def fire_b(s):
        k = s % 2
        cps = [pltpu.async_copy(v_emb.at[pidx.at[g, pl.ds(s * SUB, SUB)]],
                                pbuf.at[k, pl.ds(g * SUB, SUB)], sems[k])
               for g in range(NEG)]
        cps.append(pltpu.async_copy(v_emb.at[idx_v.at[s]], vbuf.at[k],
                                    sems[k]))
        return cps

    cps = fire_b(0)
    ocps = [None, None]
    for s in range(n_sub):
        nxt = fire_b(s + 1) if s + 1 < n_sub else None
        for cp in cps:
            cp.wait()
        k = s % 2
        if ocps[k] is not None:
            for cp in ocps[k]:
                cp.wait()

        def body(r, carry):
            dl = dbuf[s * SUB + r, :]
            one_m = jnp.float32(1.0) - dl
            for grp in range(D // 16):
                sl = pl.ds(grp * 16, 16)
                acc = pbuf[k, r, sl]
                for j in range(1, NEG):
                    acc = acc + pbuf[k, j * SUB + r, sl]
                m = acc / jnp.float32(NEG)
                psub[k, r, sl] = m
                vmsub[k, r, sl] = dl * vbuf[k, r, sl] + one_m * m
            return carry

        lax.fori_loop(0, SUB, body, jnp.int32(0))

        ocps[k] = [
            pltpu.async_copy(psub.at[k],
                             out_pmean.at[pl.ds(base + s * SUB, SUB)],
                             osems[k]),
            pltpu.async_copy(vmsub.at[k],
                             out_vmix.at[pl.ds(base + s * SUB, SUB)],
                             osems[k]),
        ]
        cps = nxt
    for cp in ocps[0] + ocps[1]:
        cp.wait()


def _sc_k2_body(nodes_u, nodes_v, protos, u_review, v_review,
                out_protos, out_urev, out_vrev,
                idx_u, idx_v, gbuf, sem0, sem1, sem2, sem3):
    wid = lax.axis_index("s") * NC + lax.axis_index("c")
    n_ch = idx_u.shape[0]
    bpw = n_ch * CHA
    base = wid * bpw

    pltpu.sync_copy(nodes_u.at[pl.ds(wid * n_ch, n_ch)], idx_u)
    pltpu.sync_copy(nodes_v.at[pl.ds(wid * n_ch, n_ch)], idx_v)

    steps = [(table, idx, out, c)
             for table, idx, out in ((u_review, idx_u, out_urev),
                                     (protos, idx_u, out_protos),
                                     (v_review, idx_v, out_vrev))
             for c in range(n_ch)]
    _plain_gathers(wid, base, steps, gbuf, (sem0, sem1), (sem2, sem3))


def _make_sc_k1(B):
    n_ch = B // (NW * CHA)
    out = jax.ShapeDtypeStruct((B, DP), jnp.float32)
    return pl.kernel(
        _sc_k1_body,
        out_type=(out,) * 3,
        mesh=plsc.VectorSubcoreMesh(core_axis_name="c", subcore_axis_name="s"),
        scratch_types=[
            pltpu.VMEM((n_ch, CHA), jnp.int32),
            pltpu.VMEM((n_ch, CHA), jnp.int32),
            pltpu.VMEM((NEG, n_ch * CHA), jnp.int32),
            pltpu.VMEM((n_ch * CHA, 16), jnp.float32),
            pltpu.VMEM((2, CHA, DP), jnp.float32),
            pltpu.VMEM((2, SUB, DP), jnp.float32),
            pltpu.VMEM((2, SUB * NEG, DP), jnp.float32),
            pltpu.VMEM((2, SUB, DP), jnp.float32),
            pltpu.VMEM((2, SUB, DP), jnp.float32),
            pltpu.SemaphoreType.DMA,
            pltpu.SemaphoreType.DMA,
            pltpu.SemaphoreType.DMA,
            pltpu.SemaphoreType.DMA,
        ],
        compiler_params=pltpu.CompilerParams(use_tc_tiling_on_sc=False),
    )


def _make_sc_k2(B):
    n_ch = B // (NW * CHA)
    out = jax.ShapeDtypeStruct((B, DP), jnp.float32)
    return pl.kernel(
        _sc_k2_body,
        out_type=(out,) * 3,
        mesh=plsc.VectorSubcoreMesh(core_axis_name="c", subcore_axis_name="s"),
        scratch_types=[
            pltpu.VMEM((n_ch, CHA), jnp.int32),
            pltpu.VMEM((n_ch, CHA), jnp.int32),
            pltpu.VMEM((2, CHA, DP), jnp.float32),
            pltpu.SemaphoreType.DMA,
            pltpu.SemaphoreType.DMA,
            pltpu.SemaphoreType.DMA,
            pltpu.SemaphoreType.DMA,
        ],
        compiler_params=pltpu.CompilerParams(use_tc_tiling_on_sc=False),
    )


def _mlp_body(uid, vmix, w1u, w1v, b1, g1, be1, w2, b2, g2, be2,
              w3, b3, g3, be3, wc, bc, out):
    def bn_relu(x, gamma, beta):
        x = jnp.maximum(x, 0.0)
        m = jnp.mean(x, axis=0, keepdims=True)
        v = jnp.mean(x * x, axis=0, keepdims=True) - m * m
        s = gamma * lax.rsqrt(v + jnp.float32(1e-5))
        return x * s + (beta - m * s)

    x1 = jnp.dot(uid[:, :D], w1u[...]) + jnp.dot(vmix[:, :D], w1v[...])
    h1 = bn_relu(x1 + b1[...], g1[...], be1[...])
    h2 = bn_relu(jnp.dot(h1, w2[...]) + b2[...], g2[...], be2[...])
    h3 = bn_relu(jnp.dot(h2, w3[...]) + b3[...], g3[...], be3[...])
    logit = jnp.sum(h3 * wc[...], axis=1, keepdims=True) + bc[...]
    out[...] = jax.nn.sigmoid(logit)


def kernel(nodes_u, nodes_v, potential_items, inter_nums, delta, global_protos,
           u_emb, v_emb, u_review, v_review,
           W1, b1, g1, be1, W2, b2, g2, be2, W3, b3, g3, be3, Wc, bc):
    B = nodes_u.shape[0]
    nodes_u2 = nodes_u.reshape(B // CHA, CHA)
    nodes_v2 = nodes_v.reshape(B // CHA, CHA)
    pot2 = potential_items.T
    delta16 = jnp.broadcast_to(delta.reshape(B, 1), (B, 16))
    zpad = jnp.broadcast_to(delta.reshape(-1)[:1] * 0.0,
                            (u_emb.shape[0], DP - D))
    pad = lambda t: jnp.concatenate([t, zpad], axis=1)

    gp_p, ur_p, vr_p = pad(global_protos), pad(u_review), pad(v_review)
    ue_p, ve_p = pad(u_emb), pad(v_emb)

    u_id_p, v_mixed_p, pmean_p = _make_sc_k1(B)(
        nodes_u2, nodes_v2, pot2, delta16, ue_p, ve_p)

    u_feats_p, u_rev_p, v_rev_p = _make_sc_k2(B)(
        nodes_u2, nodes_v2, gp_p, ur_p, vr_p)

    pred = pl.pallas_call(
        _mlp_body,
        out_shape=jax.ShapeDtypeStruct((B, 1), jnp.float32),
        compiler_params=pltpu.CompilerParams(
            vmem_limit_bytes=100 * 1024 * 1024),
    )(u_id_p, v_mixed_p, W1[:D], W1[D:], b1.reshape(1, D), g1.reshape(1, D),
      be1.reshape(1, D), W2, b2.reshape(1, D // 2), g2.reshape(1, D // 2),
      be2.reshape(1, D // 2), W3, b3.reshape(1, D // 4), g3.reshape(1, D // 4),
      be3.reshape(1, D // 4), Wc.reshape(1, D // 4), bc.reshape(1, 1))

    return (u_feats_p[:, :D], pred.reshape(B), u_id_p[:, :D],
            v_mixed_p[:, :D], u_rev_p[:, :D], v_rev_p[:, :D], pmean_p[:, :D])
